# Initial kernel scaffold; baseline (speedup 1.0000x reference)
import jax
import jax.numpy as jnp
from jax.experimental import pallas as pl


def kernel(users, series, edge_label_index):
    raise NotImplementedError("write your pallas kernel here")


# speedup vs baseline: 1.9784x; 1.9784x over previous
import functools

import jax
import jax.numpy as jnp
from jax import lax
from jax.experimental import pallas as pl
from jax.experimental.pallas import tpu as pltpu
from jax.experimental.pallas import tpu_sc as plsc

NC = 2
NS = 16
NW = NC * NS
C = 128
L = 16


@functools.lru_cache(maxsize=None)
def _build(e_pad: int, d: int):
    ew = e_pad // NW
    n_chunks = ew // C
    kd = d // L

    mesh = plsc.VectorSubcoreMesh(core_axis_name="c", subcore_axis_name="s")

    @functools.partial(
        pl.kernel,
        out_type=jax.ShapeDtypeStruct((e_pad,), jnp.float32),
        mesh=mesh,
        compiler_params=pltpu.CompilerParams(needs_layout_passes=False),
        scratch_types=[
            pltpu.VMEM((C,), jnp.int32),
            pltpu.VMEM((C,), jnp.int32),
            pltpu.VMEM((C, d), jnp.float32),
            pltpu.VMEM((C, d), jnp.float32),
            pltpu.VMEM((C,), jnp.float32),
            pltpu.SemaphoreType.DMA,
            pltpu.SemaphoreType.DMA,
        ],
    )
    def gather_dot(users_hbm, series_hbm, src_hbm, dst_hbm, out_hbm,
                   src_v, dst_v, u_v, s_v, out_v, sem_u, sem_s):
        wid = lax.axis_index("s") * NC + lax.axis_index("c")
        base = wid * ew
        rows16 = lax.iota(jnp.int32, L)

        def chunk_body(t, carry):
            off = base + t * C
            pltpu.sync_copy(src_hbm.at[pl.ds(off, C)], src_v)
            pltpu.sync_copy(dst_hbm.at[pl.ds(off, C)], dst_v)
            cu = pltpu.async_copy(users_hbm.at[src_v], u_v, sem_u)
            cs = pltpu.async_copy(series_hbm.at[dst_v], s_v, sem_s)
            cu.wait()
            cs.wait()

            def group_body(g, c2):
                row0 = g * L
                tot = jnp.zeros((L,), jnp.float32)
                for e in range(L):
                    r = row0 + e
                    acc = u_v[r, pl.ds(0, L)] * s_v[r, pl.ds(0, L)]
                    for k in range(1, kd):
                        acc = acc + (u_v[r, pl.ds(k * L, L)]
                                     * s_v[r, pl.ds(k * L, L)])
                    tot = jnp.where(rows16 == e, jnp.sum(acc), tot)
                out_v[pl.ds(row0, L)] = tot
                return c2

            lax.fori_loop(0, C // L, group_body, 0)
            pltpu.sync_copy(out_v, out_hbm.at[pl.ds(off, C)])
            return carry

        lax.fori_loop(0, n_chunks, chunk_body, 0)

    return gather_dot


def kernel(users, series, edge_label_index):
    e = edge_label_index.shape[1]
    d = users.shape[1]
    src = edge_label_index[0].astype(jnp.int32)
    dst = edge_label_index[1].astype(jnp.int32)
    e_pad = -(-e // (NW * C)) * (NW * C)
    if e_pad != e:
        src = jnp.pad(src, (0, e_pad - e))
        dst = jnp.pad(dst, (0, e_pad - e))
    out = _build(e_pad, d)(users, series, src, dst)
    return out[:e]
